# baseline (speedup 1.0000x reference)
import functools

import jax
import jax.numpy as jnp
from jax import lax
from jax.experimental import pallas as pl
from jax.experimental.pallas import tpu as pltpu
from jax.experimental.pallas import tpu_sc as plsc

HID = 2048
DFF = 768
NE = 16
NK = 8

NC = 2
NS = 16
NW = NC * NS

TM = 512
T_TOK = 8192
M8 = T_TOK * NK
M_PAD = M8 + NE * TM
NUM_M = M_PAD // TM

_TOK_W = T_TOK // NW
_CH = 16
_CT = 2


def _router_body(x_ref, gw_ref, logits_ref, wdense_ref, lm_ref, rank_ref,
                 cnt_ref, run_cnt):
    t = pl.program_id(0)
    x = x_ref[...]
    gw = gw_ref[...]
    logits = jax.lax.dot_general(
        x, gw, (((1,), (1,)), ((), ())),
        preferred_element_type=jnp.float32,
        precision=jax.lax.Precision.DEFAULT)
    logits_ref[...] = logits
    m = jnp.max(logits, axis=1, keepdims=True)
    ex = jnp.exp(logits - m)
    probs = ex / jnp.sum(ex, axis=1, keepdims=True)
    iota = jax.lax.broadcasted_iota(jnp.int32, probs.shape, 1)
    cur = probs
    wsel = jnp.zeros_like(probs)
    sel = jnp.zeros_like(probs)
    for _ in range(NK):
        mx = jnp.max(cur, axis=1, keepdims=True)
        cand = jnp.where(cur == mx, iota, NE)
        first = jnp.min(cand, axis=1, keepdims=True)
        onehot = iota == first
        wsel = jnp.where(onehot, probs, wsel)
        sel = jnp.where(onehot, 1.0, sel)
        cur = jnp.where(onehot, -jnp.inf, cur)
    wdense_ref[...] = wsel / jnp.sum(wsel, axis=1, keepdims=True)

    @pl.when(t == 0)
    def _init():
        run_cnt[...] = jnp.zeros_like(run_cnt)

    n = sel.shape[0]
    i0 = jax.lax.broadcasted_iota(jnp.int32, (n, n), 0)
    i1 = jax.lax.broadcasted_iota(jnp.int32, (n, n), 1)
    tril = jnp.where(i0 < i1, 1.0, 0.0)
    rank_local = jax.lax.dot_general(
        tril, sel, (((0,), (0,)), ((), ())),
        preferred_element_type=jnp.float32)
    rank_ref[...] = (rank_local + run_cnt[...]).astype(jnp.int32)
    new_cnt = run_cnt[...] + jnp.sum(sel, axis=0, keepdims=True)
    run_cnt[...] = new_cnt
    cnt_ref[...] = new_cnt.astype(jnp.int32)

    e0 = jax.lax.broadcasted_iota(jnp.int32, (NE, NE), 0)
    e1 = jax.lax.broadcasted_iota(jnp.int32, (NE, NE), 1)
    tril16 = jnp.where(e0 < e1, 1.0, 0.0)
    lane_rank = jax.lax.dot_general(
        sel, tril16, (((1,), (0,)), ((), ())),
        preferred_element_type=jnp.float32)
    lm_ref[...] = jnp.where(sel == 1.0, lane_rank, 15.0).astype(jnp.int32)


def _ffn_body(te_ref, xs_ref, w_ref, gwb_ref, uwb_ref, dwb_ref, ys_ref):
    del te_ref
    xs = xs_ref[...].astype(jnp.bfloat16)
    g = jax.lax.dot_general(xs, gwb_ref[0], (((1,), (1,)), ((), ())),
                            preferred_element_type=jnp.float32)
    u = jax.lax.dot_general(xs, uwb_ref[0], (((1,), (1,)), ((), ())),
                            preferred_element_type=jnp.float32)
    h = (g * jax.nn.sigmoid(g) * u).astype(jnp.bfloat16)
    y = jax.lax.dot_general(h, dwb_ref[0], (((1,), (1,)), ((), ())),
                            preferred_element_type=jnp.float32)
    ys_ref[...] = y * w_ref[...][:, :1]


@functools.lru_cache(maxsize=None)
def _sc_kernels():
    mesh = plsc.VectorSubcoreMesh(core_axis_name="c", subcore_axis_name="s")
    z16 = functools.partial(jnp.full, (16,), dtype=jnp.int32)

    @functools.partial(
        pl.kernel,
        out_type=[jax.ShapeDtypeStruct((M_PAD, HID), jnp.float32),
                  jax.ShapeDtypeStruct((M_PAD, 128), jnp.float32),
                  jax.ShapeDtypeStruct((T_TOK * NK,), jnp.int32)],
        mesh=mesh,
        compiler_params=pltpu.CompilerParams(needs_layout_passes=False),
        scratch_types=[pltpu.VMEM((_CH, NE), jnp.int32),
                       pltpu.VMEM((_CH, NE), jnp.int32),
                       pltpu.VMEM((_CH, NE), jnp.float32),
                       pltpu.VMEM((_CH, HID), jnp.float32),
                       pltpu.VMEM((NK, _CH), jnp.int32),
                       pltpu.VMEM((_CH * NK,), jnp.int32),
                       pltpu.VMEM((NE,), jnp.float32),
                       pltpu.VMEM((_CH * NK, 128), jnp.float32),
                       pltpu.SemaphoreType.DMA],
    )
    def sc_dispatch(x_hbm, pos_hbm, lm_hbm, w_hbm,
                    xs_hbm, w128_hbm, posf_hbm,
                    st_pos, st_lm, st_w, rows_v, idxt_v, posf_v, wp_v,
                    wrows_v, sem):
        wid = lax.axis_index("s") * NC + lax.axis_index("c")
        tbase = wid * _TOK_W

        def chunk(ci, carry):
            t0 = tbase + ci * _CH
            pltpu.sync_copy(pos_hbm.at[pl.ds(t0, _CH)], st_pos)
            pltpu.sync_copy(lm_hbm.at[pl.ds(t0, _CH)], st_lm)
            pltpu.sync_copy(w_hbm.at[pl.ds(t0, _CH)], st_w)
            pltpu.sync_copy(x_hbm.at[pl.ds(t0, _CH)], rows_v)
            for j in range(_CH):
                posv = st_pos[j]
                lmv = st_lm[j]
                wv = st_w[j]
                valid = lmv < NK
                plsc.store_scatter(posf_v, [lmv + (j * NK)], posv, mask=valid)
                plsc.store_scatter(wp_v, [lmv], wv, mask=valid)
                for k in range(NK):
                    spl = plsc.load_gather(wp_v, [z16(k)])
                    for s in range(8):
                        wrows_v[j * NK + k, pl.ds(s * 16, 16)] = spl
            lane = lax.iota(jnp.int32, 16)
            for k in range(NK):
                idxt_v[k, :] = plsc.load_gather(posf_v, [lane * NK + k])
            pltpu.sync_copy(posf_v, posf_hbm.at[pl.ds(t0 * NK, _CH * NK)])
            hw = pltpu.async_copy(wrows_v, w128_hbm.at[posf_v], sem)
            hx = [pltpu.async_copy(rows_v, xs_hbm.at[idxt_v.at[k]], sem)
                  for k in range(NK)]
            hw.wait()
            for h in hx:
                h.wait()
            return carry

        lax.fori_loop(0, _TOK_W // _CH, chunk, 0)

    @functools.partial(
        pl.kernel,
        out_type=jax.ShapeDtypeStruct((T_TOK, HID), jnp.float32),
        mesh=mesh,
        scratch_types=[pltpu.VMEM((_CT * NK,), jnp.int32),
                       pltpu.VMEM((_CT * NK,), jnp.int32),
                       pltpu.VMEM((_CT * NK, HID), jnp.float32),
                       pltpu.VMEM((_CT * NK, HID), jnp.float32),
                       pltpu.VMEM((_CT, HID), jnp.float32),
                       pltpu.SemaphoreType.DMA,
                       pltpu.SemaphoreType.DMA],
    )
    def sc_combine(ys_hbm, pos_hbm, out_hbm, idx_v0, idx_v1, rows_v0,
                   rows_v1, outc_v, sem0, sem1):
        wid = lax.axis_index("s") * NC + lax.axis_index("c")
        tbase = wid * _TOK_W

        def compute(rows_v, t0):
            def cbody(c, carry2):
                o = c * 16
                for j in range(_CT):
                    acc = rows_v[j * NK, pl.ds(o, 16)]
                    for k in range(1, NK):
                        acc = acc + rows_v[j * NK + k, pl.ds(o, 16)]
                    outc_v[j, pl.ds(o, 16)] = acc
                return carry2

            lax.fori_loop(0, HID // 16, cbody, 0)
            pltpu.sync_copy(outc_v, out_hbm.at[pl.ds(t0, _CT)])

        def chunk(ci, carry):
            t0 = tbase + ci * 2 * _CT
            t1 = t0 + _CT
            pltpu.sync_copy(pos_hbm.at[pl.ds(t0 * NK, _CT * NK)], idx_v0)
            h0 = pltpu.async_copy(ys_hbm.at[idx_v0], rows_v0, sem0)
            pltpu.sync_copy(pos_hbm.at[pl.ds(t1 * NK, _CT * NK)], idx_v1)
            h1 = pltpu.async_copy(ys_hbm.at[idx_v1], rows_v1, sem1)
            h0.wait()
            compute(rows_v0, t0)
            h1.wait()
            compute(rows_v1, t1)
            return carry

        lax.fori_loop(0, _TOK_W // (2 * _CT), chunk, 0)

    return sc_dispatch, sc_combine


def kernel(hidden_states, gate_w, gate_ws, up_ws, down_ws):
    bsz, seq, hd = hidden_states.shape
    T = bsz * seq
    x = hidden_states.reshape(T, hd)

    TMR = 1024
    logits, wdense, lm, rank_g, cnt = pl.pallas_call(
        _router_body,
        grid=(T // TMR,),
        in_specs=[pl.BlockSpec((TMR, HID), lambda t: (t, 0)),
                  pl.BlockSpec((NE, HID), lambda t: (0, 0))],
        out_specs=[pl.BlockSpec((TMR, NE), lambda t: (t, 0)),
                   pl.BlockSpec((TMR, NE), lambda t: (t, 0)),
                   pl.BlockSpec((TMR, NE), lambda t: (t, 0)),
                   pl.BlockSpec((TMR, NE), lambda t: (t, 0)),
                   pl.BlockSpec((1, NE), lambda t: (0, 0))],
        out_shape=[jax.ShapeDtypeStruct((T, NE), jnp.float32),
                   jax.ShapeDtypeStruct((T, NE), jnp.float32),
                   jax.ShapeDtypeStruct((T, NE), jnp.int32),
                   jax.ShapeDtypeStruct((T, NE), jnp.int32),
                   jax.ShapeDtypeStruct((1, NE), jnp.int32)],
        scratch_shapes=[pltpu.VMEM((1, NE), jnp.float32)],
    )(x, gate_w)

    cnt16 = cnt.reshape(NE)
    pad_cnt = ((cnt16 + TM - 1) // TM) * TM
    ends = jnp.cumsum(pad_cnt)
    off = ends - pad_cnt
    pos16 = off[None, :] + rank_g
    starts = jnp.arange(NUM_M, dtype=jnp.int32) * TM
    tile_expert = jnp.sum(
        (ends[None, :] <= starts[:, None]).astype(jnp.int32), axis=1)
    tile_expert = jnp.minimum(tile_expert, NE - 1)

    sc_dispatch, sc_combine = _sc_kernels()
    xs, w128, pos8f = sc_dispatch(x, pos16, lm, wdense)

    gwb = gate_ws.astype(jnp.bfloat16)
    uwb = up_ws.astype(jnp.bfloat16)
    dwb = down_ws.astype(jnp.bfloat16)

    grid_spec = pltpu.PrefetchScalarGridSpec(
        num_scalar_prefetch=1,
        grid=(NUM_M,),
        in_specs=[
            pl.BlockSpec((TM, HID), lambda i, te: (i, 0)),
            pl.BlockSpec((TM, 128), lambda i, te: (i, 0)),
            pl.BlockSpec((1, DFF, HID), lambda i, te: (te[i], 0, 0)),
            pl.BlockSpec((1, DFF, HID), lambda i, te: (te[i], 0, 0)),
            pl.BlockSpec((1, HID, DFF), lambda i, te: (te[i], 0, 0)),
        ],
        out_specs=pl.BlockSpec((TM, HID), lambda i, te: (i, 0)),
    )
    ys = pl.pallas_call(
        _ffn_body,
        grid_spec=grid_spec,
        out_shape=jax.ShapeDtypeStruct((M_PAD, HID), jnp.float32),
    )(tile_expert, xs, w128, gwb, uwb, dwb)

    final = sc_combine(ys, pos8f)
    return final.reshape(bsz, seq, hd), logits
